# baseline (speedup 1.0000x reference)
import functools

import jax
import jax.numpy as jnp
from jax import lax
from jax.experimental import pallas as pl
from jax.experimental.pallas import tpu as pltpu
from jax.experimental.pallas import tpu_sc as plsc

N = 10000
E = 160000
D = 128
NC = 2
NT = 16
E_PAD = 163840
EPT = E_PAD // NT
CH = 32
NCH = EPT // CH
NROW = EPT // 128
WPT = E_PAD // (NC * NT)
WCH = WPT // CH
N_PAD = 10240
RPT = N_PAD // NT
HQ = 16
CPH = 4 * HQ
RING = 8


def _sc_aggregate(x2, src, dst, cnt, zeros, with_w):
    mesh = plsc.VectorSubcoreMesh(core_axis_name="c", subcore_axis_name="s")
    out_rows = (4 if with_w else 2) * N_PAD

    @functools.partial(
        pl.kernel,
        out_type=jax.ShapeDtypeStruct((out_rows, D), jnp.float32),
        mesh=mesh,
        scratch_types=[
            pltpu.VMEM((HQ, 128), jnp.int32),
            pltpu.VMEM((HQ, 128), jnp.int32),
            pltpu.VMEM((HQ, 128), jnp.float32),
            [pltpu.VMEM((CH,), jnp.int32)] * RING,
            [pltpu.VMEM((CH,), jnp.int32)] * RING,
            [pltpu.VMEM((CH, D), jnp.float32)] * RING,
            pltpu.VMEM_SHARED((N_PAD, D), jnp.float32),
            [pltpu.SemaphoreType.DMA] * RING,
            [pltpu.SemaphoreType.DMA] * RING,
        ],
    )
    def agg(x2_hbm, src_hbm, dst_hbm, cnt_hbm, z_hbm, out_hbm,
            src_s, dst_s, cnt_s, idxs, dsts, bufs, acc, gsems, ssems):
        c = lax.axis_index("c")
        s = lax.axis_index("s")
        pltpu.sync_copy(z_hbm, acc.at[pl.ds(s * RPT, RPT)])
        plsc.subcore_barrier()

        cvec = jnp.full((16,), c, dtype=jnp.int32)

        def build_idx(ch, idx_ref):
            row = ch >> 2
            cb = (ch & 3) * CH
            for g in range(CH // 16):
                idx_ref[pl.ds(g * 16, 16)] = (
                    src_s[row, pl.ds(cb + g * 16, 16)] * 2 + cvec)

        def copy_dst(ch, dref):
            row = ch >> 2
            cb = (ch & 3) * CH
            for g in range(CH // 16):
                dref[pl.ds(g * 16, 16)] = dst_s[row, pl.ds(cb + g * 16, 16)]

        def scale(ch, buf):
            row = ch >> 2
            cb = (ch & 3) * CH

            def group(g, carry):
                c16 = cnt_s[row, pl.ds(cb + g * 16, 16)]
                base = g * 16
                for j in range(16):
                    cvv = jnp.full((16,), c16[j], dtype=jnp.float32)
                    for f in range(D // 16):
                        fsl = pl.ds(f * 16, 16)
                        buf[base + j, fsl] = buf[base + j, fsl] * cvv
                return carry
            lax.fori_loop(0, CH // 16, group, 0)

        def gather_wait(k):
            pltpu.make_async_copy(x2_hbm.at[idxs[k]], bufs[k], gsems[k]).wait()

        def scatter_wait(k):
            pltpu.make_async_copy(bufs[k], acc.at[dsts[k]], ssems[k]).wait()

        def half(hh, carry0):
            hb = s * NROW + hh * HQ
            pltpu.sync_copy(src_hbm.at[pl.ds(hb, HQ)], src_s)
            pltpu.sync_copy(dst_hbm.at[pl.ds(hb, HQ)], dst_s)
            pltpu.sync_copy(cnt_hbm.at[pl.ds(hb, HQ)], cnt_s)
            for k in range(RING - 1):
                build_idx(k, idxs[k])
                pltpu.async_copy(x2_hbm.at[idxs[k]], bufs[k], gsems[k])

            def quad(p, carry):
                for k in range(RING):
                    j = RING * p + k
                    gather_wait(k)
                    scale(j, bufs[k])
                    copy_dst(j, dsts[k])
                    pltpu.async_copy(bufs[k], acc.at[dsts[k]], ssems[k],
                                     add=True)
                    rb = (k + RING - 1) % RING
                    nxt = jnp.minimum(j + RING - 1, CPH - 1)
                    build_idx(nxt, idxs[rb])
                    if k == 0:
                        @pl.when(p > 0)
                        def _():
                            scatter_wait(rb)
                    else:
                        scatter_wait(rb)
                    pltpu.async_copy(x2_hbm.at[idxs[rb]], bufs[rb],
                                     gsems[rb])
                return carry

            lax.fori_loop(0, CPH // RING, quad, 0)
            for k in range(RING - 1):
                gather_wait(k)
            scatter_wait(RING - 1)
            return carry0

        lax.fori_loop(0, NROW // HQ, half, 0)
        plsc.subcore_barrier()
        pltpu.sync_copy(acc.at[pl.ds(s * RPT, RPT)],
                        out_hbm.at[pl.ds(c * N_PAD + s * RPT, RPT)])

        if with_w:
            plsc.subcore_barrier()
            pltpu.sync_copy(z_hbm, acc.at[pl.ds(s * RPT, RPT)])
            pltpu.sync_copy(z_hbm.at[pl.ds(0, CH)], bufs[0])
            wid = s * NC + c
            plsc.subcore_barrier()
            lane0 = jnp.where(lax.iota(jnp.int32, 16) == 0,
                              jnp.full((16,), 1.0, dtype=jnp.float32),
                              jnp.full((16,), 0.0, dtype=jnp.float32))

            def wchunk(i, carry):
                row = i >> 2
                cb = (i & 3) * CH

                def group(g, carry2):
                    c16 = cnt_s[row, pl.ds(cb + g * 16, 16)]
                    base = g * 16
                    for j in range(16):
                        bufs[0][base + j, pl.ds(0, 16)] = lane0 * jnp.full(
                            (16,), c16[j], dtype=jnp.float32)
                    return carry2
                lax.fori_loop(0, CH // 16, group, 0)
                copy_dst(i, dsts[0])
                pltpu.sync_copy(bufs[0], acc.at[dsts[0]], add=True)
                return carry

            def wstint(t, carry):
                wb = wid * (E_PAD // (NC * NT * 128)) + t * 8
                pltpu.sync_copy(dst_hbm.at[pl.ds(wb, 8)],
                                dst_s.at[pl.ds(0, 8)])
                pltpu.sync_copy(cnt_hbm.at[pl.ds(wb, 8)],
                                cnt_s.at[pl.ds(0, 8)])
                lax.fori_loop(0, 32, wchunk, 0)
                return carry

            lax.fori_loop(0, 5, wstint, 0)
            plsc.subcore_barrier()
            pltpu.sync_copy(
                acc.at[pl.ds(s * RPT, RPT)],
                out_hbm.at[pl.ds((2 + c) * N_PAD + s * RPT, RPT)])

    return agg(x2, src, dst, cnt, zeros)


def _tc_layer(a0, a1, w0, w1, h, wn0, wn1, wh, b):
    br = 1000

    def body(a0_r, a1_r, w0_r, w1_r, h_r, wn0_r, wn1_r, wh_r, b_r, o_r):
        w = w0_r[:, :1] + w1_r[:, :1]
        inv = 1.0 / jnp.maximum(w, 1.0)
        n0 = a0_r[...] * inv
        n1 = a1_r[...] * inv
        z = (jnp.dot(n0, wn0_r[...], preferred_element_type=jnp.float32)
             + jnp.dot(n1, wn1_r[...], preferred_element_type=jnp.float32)
             + jnp.dot(h_r[...], wh_r[...], preferred_element_type=jnp.float32)
             + b_r[...])
        z = jnp.maximum(z, 0.0)
        ssum = jnp.sum(z * z, axis=1, keepdims=True)
        o_r[...] = z * lax.rsqrt(jnp.where(ssum == 0.0, 1.0, ssum))

    return pl.pallas_call(
        body,
        grid=(N // br,),
        in_specs=[
            pl.BlockSpec((br, D), lambda i: (i, 0)),
            pl.BlockSpec((br, D), lambda i: (i, 0)),
            pl.BlockSpec((br, D), lambda i: (i, 0)),
            pl.BlockSpec((br, D), lambda i: (i, 0)),
            pl.BlockSpec((br, 2 * D), lambda i: (i, 0)),
            pl.BlockSpec((D, 2 * D), lambda i: (0, 0)),
            pl.BlockSpec((D, 2 * D), lambda i: (0, 0)),
            pl.BlockSpec((2 * D, 2 * D), lambda i: (0, 0)),
            pl.BlockSpec((1, 2 * D), lambda i: (0, 0)),
        ],
        out_specs=pl.BlockSpec((br, 2 * D), lambda i: (i, 0)),
        out_shape=jax.ShapeDtypeStruct((N, 2 * D), jnp.float32),
    )(a0, a1, w0, w1, h, wn0, wn1, wh, b.reshape(1, 2 * D))


def kernel(x, edge_index, edge_count, W1, b1, W2, b2):
    src = edge_index[0].astype(jnp.int32)
    dst = edge_index[1].astype(jnp.int32)
    cnt = edge_count.astype(jnp.float32)
    pad = E_PAD - E
    src_p = jnp.concatenate([src, jnp.zeros((pad,), jnp.int32)])
    dst_p = jnp.concatenate([dst, jnp.zeros((pad,), jnp.int32)])
    cnt_p = jnp.concatenate([cnt, jnp.zeros((pad,), jnp.float32)])
    src_p = src_p.reshape(E_PAD // 128, 128)
    dst_p = dst_p.reshape(E_PAD // 128, 128)
    cnt_p = cnt_p.reshape(E_PAD // 128, 128)
    zeros = jnp.zeros((RPT, D), jnp.float32)

    def layer(h, W, b, w0, w1):
        agg = _sc_aggregate(h.reshape(2 * N, D), src_p, dst_p, cnt_p, zeros,
                            with_w=w0 is None)
        if w0 is None:
            w0 = agg[2 * N_PAD:2 * N_PAD + N]
            w1 = agg[3 * N_PAD:3 * N_PAD + N]
        out = _tc_layer(agg[:N], agg[N_PAD:N_PAD + N], w0, w1, h,
                        W[:D], W[D:2 * D], W[2 * D:], b)
        return out, w0, w1

    h1, w0, w1 = layer(x, W1, b1, None, None)
    h2, _, _ = layer(h1, W2, b2, w0, w1)
    return h2


# speedup vs baseline: 1.0880x; 1.0880x over previous
import functools

import jax
import jax.numpy as jnp
from jax import lax
from jax.experimental import pallas as pl
from jax.experimental.pallas import tpu as pltpu
from jax.experimental.pallas import tpu_sc as plsc

N = 10000
E = 160000
D = 128
NC = 2
NT = 16
E_PAD = 163840
EPT = E_PAD // NT
CH = 64
NCH = EPT // CH
NROW = EPT // 128
WPT = E_PAD // (NC * NT)
WCH = WPT // CH
N_PAD = 10240
RPT = N_PAD // NT
HQ = 40
CPH = 2 * HQ
RING = 4


def _sc_aggregate(x2, src, dst, cnt, zeros, with_w):
    mesh = plsc.VectorSubcoreMesh(core_axis_name="c", subcore_axis_name="s")
    out_rows = (4 if with_w else 2) * N_PAD

    @functools.partial(
        pl.kernel,
        out_type=jax.ShapeDtypeStruct((out_rows, D), jnp.float32),
        mesh=mesh,
        scratch_types=[
            pltpu.VMEM((HQ, 128), jnp.int32),
            pltpu.VMEM((HQ, 128), jnp.int32),
            pltpu.VMEM((HQ, 128), jnp.float32),
            [pltpu.VMEM((CH,), jnp.int32)] * RING,
            [pltpu.VMEM((CH,), jnp.int32)] * RING,
            [pltpu.VMEM((CH, D), jnp.float32)] * RING,
            pltpu.VMEM_SHARED((N_PAD, D), jnp.float32),
            [pltpu.SemaphoreType.DMA] * RING,
            [pltpu.SemaphoreType.DMA] * RING,
        ],
    )
    def agg(x2_hbm, src_hbm, dst_hbm, cnt_hbm, z_hbm, out_hbm,
            src_s, dst_s, cnt_s, idxs, dsts, bufs, acc, gsems, ssems):
        c = lax.axis_index("c")
        s = lax.axis_index("s")
        pltpu.sync_copy(z_hbm, acc.at[pl.ds(s * RPT, RPT)])
        plsc.subcore_barrier()

        cvec = jnp.full((16,), c, dtype=jnp.int32)

        def build_idx(ch, idx_ref):
            row = ch >> 1
            cb = (ch & 1) * CH
            for g in range(CH // 16):
                idx_ref[pl.ds(g * 16, 16)] = (
                    src_s[row, pl.ds(cb + g * 16, 16)] * 2 + cvec)

        def copy_dst(ch, dref):
            row = ch >> 1
            cb = (ch & 1) * CH
            for g in range(CH // 16):
                dref[pl.ds(g * 16, 16)] = dst_s[row, pl.ds(cb + g * 16, 16)]

        def scale(ch, buf):
            row = ch >> 1
            cb = (ch & 1) * CH

            def group(g, carry):
                c16 = cnt_s[row, pl.ds(cb + g * 16, 16)]
                base = g * 16
                for j in range(16):
                    cvv = jnp.full((16,), c16[j], dtype=jnp.float32)
                    for f in range(D // 16):
                        fsl = pl.ds(f * 16, 16)
                        buf[base + j, fsl] = buf[base + j, fsl] * cvv
                return carry
            lax.fori_loop(0, CH // 16, group, 0)

        def gather_wait(k):
            pltpu.make_async_copy(x2_hbm.at[idxs[k]], bufs[k], gsems[k]).wait()

        def scatter_wait(k):
            pltpu.make_async_copy(bufs[k], acc.at[dsts[k]], ssems[k]).wait()

        def half(hh, carry0):
            hb = s * NROW + hh * HQ
            pltpu.sync_copy(src_hbm.at[pl.ds(hb, HQ)], src_s)
            pltpu.sync_copy(dst_hbm.at[pl.ds(hb, HQ)], dst_s)
            pltpu.sync_copy(cnt_hbm.at[pl.ds(hb, HQ)], cnt_s)
            for k in range(RING - 1):
                build_idx(k, idxs[k])
                pltpu.async_copy(x2_hbm.at[idxs[k]], bufs[k], gsems[k])

            def quad(p, carry):
                for k in range(RING):
                    j = RING * p + k
                    gather_wait(k)
                    scale(j, bufs[k])
                    copy_dst(j, dsts[k])
                    pltpu.async_copy(bufs[k], acc.at[dsts[k]], ssems[k],
                                     add=True)
                    rb = (k + RING - 1) % RING
                    nxt = jnp.minimum(j + RING - 1, CPH - 1)
                    build_idx(nxt, idxs[rb])
                    if k == 0:
                        @pl.when(p > 0)
                        def _():
                            scatter_wait(rb)
                    else:
                        scatter_wait(rb)
                    pltpu.async_copy(x2_hbm.at[idxs[rb]], bufs[rb],
                                     gsems[rb])
                return carry

            lax.fori_loop(0, CPH // RING, quad, 0)
            for k in range(RING - 1):
                gather_wait(k)
            scatter_wait(RING - 1)
            return carry0

        lax.fori_loop(0, NROW // HQ, half, 0)
        plsc.subcore_barrier()
        pltpu.sync_copy(acc.at[pl.ds(s * RPT, RPT)],
                        out_hbm.at[pl.ds(c * N_PAD + s * RPT, RPT)])

        if with_w:
            plsc.subcore_barrier()
            pltpu.sync_copy(z_hbm, acc.at[pl.ds(s * RPT, RPT)])
            pltpu.sync_copy(z_hbm.at[pl.ds(0, CH)], bufs[0])
            wid = s * NC + c
            pltpu.sync_copy(dst_hbm.at[pl.ds(wid * HQ, HQ)], dst_s)
            pltpu.sync_copy(cnt_hbm.at[pl.ds(wid * HQ, HQ)], cnt_s)
            plsc.subcore_barrier()
            lane0 = jnp.where(lax.iota(jnp.int32, 16) == 0,
                              jnp.full((16,), 1.0, dtype=jnp.float32),
                              jnp.full((16,), 0.0, dtype=jnp.float32))

            def wchunk(i, carry):
                row = i >> 1
                cb = (i & 1) * CH

                def group(g, carry2):
                    c16 = cnt_s[row, pl.ds(cb + g * 16, 16)]
                    base = g * 16
                    for j in range(16):
                        bufs[0][base + j, pl.ds(0, 16)] = lane0 * jnp.full(
                            (16,), c16[j], dtype=jnp.float32)
                    return carry2
                lax.fori_loop(0, CH // 16, group, 0)
                copy_dst(i, dsts[0])
                pltpu.sync_copy(bufs[0], acc.at[dsts[0]], add=True)
                return carry

            lax.fori_loop(0, CPH, wchunk, 0)
            plsc.subcore_barrier()
            pltpu.sync_copy(
                acc.at[pl.ds(s * RPT, RPT)],
                out_hbm.at[pl.ds((2 + c) * N_PAD + s * RPT, RPT)])

    return agg(x2, src, dst, cnt, zeros)


def _tc_layer(a0, a1, w0, w1, h, wn0, wn1, wh, b):
    br = 1000

    def body(a0_r, a1_r, w0_r, w1_r, h_r, wn0_r, wn1_r, wh_r, b_r, o_r):
        w = w0_r[:, :1] + w1_r[:, :1]
        inv = 1.0 / jnp.maximum(w, 1.0)
        n0 = a0_r[...] * inv
        n1 = a1_r[...] * inv
        z = (jnp.dot(n0, wn0_r[...], preferred_element_type=jnp.float32)
             + jnp.dot(n1, wn1_r[...], preferred_element_type=jnp.float32)
             + jnp.dot(h_r[...], wh_r[...], preferred_element_type=jnp.float32)
             + b_r[...])
        z = jnp.maximum(z, 0.0)
        ssum = jnp.sum(z * z, axis=1, keepdims=True)
        o_r[...] = z * lax.rsqrt(jnp.where(ssum == 0.0, 1.0, ssum))

    return pl.pallas_call(
        body,
        grid=(N // br,),
        in_specs=[
            pl.BlockSpec((br, D), lambda i: (i, 0)),
            pl.BlockSpec((br, D), lambda i: (i, 0)),
            pl.BlockSpec((br, D), lambda i: (i, 0)),
            pl.BlockSpec((br, D), lambda i: (i, 0)),
            pl.BlockSpec((br, 2 * D), lambda i: (i, 0)),
            pl.BlockSpec((D, 2 * D), lambda i: (0, 0)),
            pl.BlockSpec((D, 2 * D), lambda i: (0, 0)),
            pl.BlockSpec((2 * D, 2 * D), lambda i: (0, 0)),
            pl.BlockSpec((1, 2 * D), lambda i: (0, 0)),
        ],
        out_specs=pl.BlockSpec((br, 2 * D), lambda i: (i, 0)),
        out_shape=jax.ShapeDtypeStruct((N, 2 * D), jnp.float32),
    )(a0, a1, w0, w1, h, wn0, wn1, wh, b.reshape(1, 2 * D))


def kernel(x, edge_index, edge_count, W1, b1, W2, b2):
    src = edge_index[0].astype(jnp.int32)
    dst = edge_index[1].astype(jnp.int32)
    cnt = edge_count.astype(jnp.float32)
    pad = E_PAD - E
    src_p = jnp.concatenate([src, jnp.zeros((pad,), jnp.int32)])
    dst_p = jnp.concatenate([dst, jnp.zeros((pad,), jnp.int32)])
    cnt_p = jnp.concatenate([cnt, jnp.zeros((pad,), jnp.float32)])
    src_p = src_p.reshape(E_PAD // 128, 128)
    dst_p = dst_p.reshape(E_PAD // 128, 128)
    cnt_p = cnt_p.reshape(E_PAD // 128, 128)
    zeros = jnp.zeros((RPT, D), jnp.float32)

    def layer(h, W, b, w0, w1):
        agg = _sc_aggregate(h.reshape(2 * N, D), src_p, dst_p, cnt_p, zeros,
                            with_w=w0 is None)
        if w0 is None:
            w0 = agg[2 * N_PAD:2 * N_PAD + N]
            w1 = agg[3 * N_PAD:3 * N_PAD + N]
        out = _tc_layer(agg[:N], agg[N_PAD:N_PAD + N], w0, w1, h,
                        W[:D], W[D:2 * D], W[2 * D:], b)
        return out, w0, w1

    h1, w0, w1 = layer(x, W1, b1, None, None)
    h2, _, _ = layer(h1, W2, b2, w0, w1)
    return h2
